# baseline (speedup 1.0000x reference)
import functools

import jax
import jax.numpy as jnp
from jax import lax
from jax.experimental import pallas as pl
from jax.experimental.pallas import tpu as pltpu
from jax.experimental.pallas import tpu_sc as plsc

N = 10000
E = 320000
IN_DIM = 128
HIDDEN = 64
EMBED = 32

NC = 2
NS = 16
L = 16

BURST = 40
CHUNK = 400
OPS = CHUNK // BURST
EDGES_PER_WORKER = E // (NC * NS)
BURSTS_PER_WORKER = EDGES_PER_WORKER // BURST
CHUNKS = EDGES_PER_WORKER // CHUNK
NPAD = 10240
NODES_PER_TILE = NPAD // NS


def _make_sc_aggregate(d):
  mesh = plsc.VectorSubcoreMesh(core_axis_name="c", subcore_axis_name="s")
  nbuf = 2 if d > 32 else 4
  outer = CHUNKS // nbuf
  tail = CHUNKS - outer * nbuf

  @functools.partial(
      pl.kernel,
      out_type=jax.ShapeDtypeStruct((NC, NPAD, d), jnp.float32),
      mesh=mesh,
      scratch_types=[
          pltpu.VMEM_SHARED((NPAD, d), jnp.float32),
          pltpu.VMEM((BURSTS_PER_WORKER, BURST), jnp.int32),
          pltpu.VMEM((BURSTS_PER_WORKER, BURST), jnp.int32),
          pltpu.VMEM((EDGES_PER_WORKER,), jnp.float32),
          pltpu.VMEM((nbuf, CHUNK, d), jnp.float32),
          pltpu.SemaphoreType.DMA,
          pltpu.SemaphoreType.DMA,
          pltpu.SemaphoreType.DMA,
      ],
      compiler_params=pltpu.CompilerParams(
          needs_layout_passes=False, use_tc_tiling_on_sc=False),
  )
  def agg(h_hbm, src_hbm, dst_hbm, w_hbm, out_hbm,
          acc, src_v, dst_v, w_v, rows_v, lsem, gsem, ssem):
    c = lax.axis_index("c")
    s = lax.axis_index("s")
    wid = c * NS + s

    lcps = [
        pltpu.async_copy(src_hbm.at[wid], src_v, lsem),
        pltpu.async_copy(dst_hbm.at[wid], dst_v, lsem),
        pltpu.async_copy(w_hbm.at[wid], w_v, lsem),
    ]

    def zero_row(i, carry):
      for j in range(d // L):
        rows_v[0, i, pl.ds(j * L, L)] = jnp.zeros((L,), jnp.float32)
      return carry
    lax.fori_loop(0, CHUNK, zero_row, 0)
    base_n = s * NODES_PER_TILE
    done = 0
    zcps = []
    while done < NODES_PER_TILE:
      step = min(CHUNK, NODES_PER_TILE - done)
      zcps.append(pltpu.async_copy(rows_v.at[0, pl.ds(0, step)],
                                   acc.at[pl.ds(base_n + done, step)], gsem))
      done += step
    for cp in zcps + lcps:
      cp.wait()
    plsc.subcore_barrier()

    def do_round(base_c, nbuf):
      gcps = []
      for b in range(nbuf):
        for j in range(OPS):
          row = (base_c + b) * OPS + j
          gcps.append(
              pltpu.async_copy(h_hbm.at[src_v.at[row]],
                               rows_v.at[b, pl.ds(j * BURST, BURST)], gsem))
      scps = []
      for b in range(nbuf):
        for j in range(OPS):
          gcps.pop(0).wait()
        base_e = (base_c + b) * CHUNK
        def mul_row(i, b=b, base_e=base_e):
          wb = plsc.load_gather(
              w_v, [jnp.full((L,), 0, jnp.int32) + base_e + i])
          for jj in range(d // L):
            rows_v[b, i, pl.ds(jj * L, L)] = (
                rows_v[b, i, pl.ds(jj * L, L)] * wb)
        plsc.parallel_loop(0, CHUNK, unroll=4)(mul_row)
        for j in range(OPS):
          row = (base_c + b) * OPS + j
          scps.append(
              pltpu.async_copy(rows_v.at[b, pl.ds(j * BURST, BURST)],
                               acc.at[dst_v.at[row]], ssem, add=True))
      for cp in scps:
        cp.wait()

    def outer_body(o, carry):
      do_round(o * nbuf, nbuf)
      return carry

    lax.fori_loop(0, outer, outer_body, 0)
    if tail:
      do_round(outer * nbuf, tail)
    plsc.subcore_barrier()
    pltpu.sync_copy(acc.at[pl.ds(base_n, NODES_PER_TILE)],
                    out_hbm.at[c, pl.ds(base_n, NODES_PER_TILE)])

  return agg


_sc_agg_hidden = _make_sc_aggregate(HIDDEN)
_sc_agg_embed = _make_sc_aggregate(EMBED)


def _mm_body(x_ref, w_ref, o_ref):
  o_ref[...] = jnp.dot(x_ref[...], w_ref[...],
                       preferred_element_type=jnp.float32)


def _fuse_mm_body(p_ref, b_ref, w_ref, o_ref):
  h = jnp.maximum(p_ref[0, :N] + p_ref[1, :N] + b_ref[...], 0.0)
  o_ref[...] = jnp.dot(h, w_ref[...], preferred_element_type=jnp.float32)


def _combine_body(p_ref, b_ref, o_ref):
  o_ref[...] = p_ref[0, :N] + p_ref[1, :N] + b_ref[...]


@jax.jit
def kernel(x, edge_index, edge_weight, W1, b1, W2, b2):
  src3d = edge_index[0].reshape(NC * NS, BURSTS_PER_WORKER, BURST)
  dst3d = edge_index[1].reshape(NC * NS, BURSTS_PER_WORKER, BURST)
  w2d = edge_weight.reshape(NC * NS, EDGES_PER_WORKER)

  h1 = pl.pallas_call(
      _mm_body,
      out_shape=jax.ShapeDtypeStruct((N, HIDDEN), jnp.float32),
  )(x, W1)

  p1 = _sc_agg_hidden(h1, src3d, dst3d, w2d)

  h2 = pl.pallas_call(
      _fuse_mm_body,
      out_shape=jax.ShapeDtypeStruct((N, EMBED), jnp.float32),
  )(p1, b1.reshape(1, HIDDEN), W2)

  p2 = _sc_agg_embed(h2, src3d, dst3d, w2d)

  out = pl.pallas_call(
      _combine_body,
      out_shape=jax.ShapeDtypeStruct((N, EMBED), jnp.float32),
  )(p2, b2.reshape(1, EMBED))
  return out


# speedup vs baseline: 1.0494x; 1.0494x over previous
import functools

import jax
import jax.numpy as jnp
from jax import lax
from jax.experimental import pallas as pl
from jax.experimental.pallas import tpu as pltpu
from jax.experimental.pallas import tpu_sc as plsc

N = 10000
E = 320000
IN_DIM = 128
HIDDEN = 64
EMBED = 32

NC = 2
NS = 16
L = 16

BURST = 40
CHUNK = 200
OPS = CHUNK // BURST
EDGES_PER_WORKER = E // (NC * NS)
BURSTS_PER_WORKER = EDGES_PER_WORKER // BURST
CHUNKS = EDGES_PER_WORKER // CHUNK
NPAD = 10240
NODES_PER_TILE = NPAD // NS


def _make_sc_aggregate(d):
  mesh = plsc.VectorSubcoreMesh(core_axis_name="c", subcore_axis_name="s")
  nbuf = 4 if d > 32 else 8
  outer = CHUNKS // nbuf
  tail = CHUNKS - outer * nbuf

  @functools.partial(
      pl.kernel,
      out_type=jax.ShapeDtypeStruct((NC, NPAD, d), jnp.float32),
      mesh=mesh,
      scratch_types=[
          pltpu.VMEM_SHARED((NPAD, d), jnp.float32),
          pltpu.VMEM((BURSTS_PER_WORKER, BURST), jnp.int32),
          pltpu.VMEM((BURSTS_PER_WORKER, BURST), jnp.int32),
          pltpu.VMEM((EDGES_PER_WORKER,), jnp.float32),
          pltpu.VMEM((nbuf, CHUNK, d), jnp.float32),
          pltpu.SemaphoreType.DMA,
          pltpu.SemaphoreType.DMA,
          pltpu.SemaphoreType.DMA,
      ],
      compiler_params=pltpu.CompilerParams(
          needs_layout_passes=False, use_tc_tiling_on_sc=False),
  )
  def agg(h_hbm, src_hbm, dst_hbm, w_hbm, out_hbm,
          acc, src_v, dst_v, w_v, rows_v, lsem, gsem, ssem):
    c = lax.axis_index("c")
    s = lax.axis_index("s")
    wid = c * NS + s

    lcps = [
        pltpu.async_copy(src_hbm.at[wid], src_v, lsem),
        pltpu.async_copy(dst_hbm.at[wid], dst_v, lsem),
        pltpu.async_copy(w_hbm.at[wid], w_v, lsem),
    ]

    def zero_row(i, carry):
      for j in range(d // L):
        rows_v[0, i, pl.ds(j * L, L)] = jnp.zeros((L,), jnp.float32)
      return carry
    lax.fori_loop(0, CHUNK, zero_row, 0)
    base_n = s * NODES_PER_TILE
    done = 0
    zcps = []
    while done < NODES_PER_TILE:
      step = min(CHUNK, NODES_PER_TILE - done)
      zcps.append(pltpu.async_copy(rows_v.at[0, pl.ds(0, step)],
                                   acc.at[pl.ds(base_n + done, step)], gsem))
      done += step
    for cp in zcps + lcps:
      cp.wait()
    plsc.subcore_barrier()

    def do_round(base_c, nbuf):
      gcps = []
      for b in range(nbuf):
        for j in range(OPS):
          row = (base_c + b) * OPS + j
          gcps.append(
              pltpu.async_copy(h_hbm.at[src_v.at[row]],
                               rows_v.at[b, pl.ds(j * BURST, BURST)], gsem))
      scps = []
      for b in range(nbuf):
        for j in range(OPS):
          gcps.pop(0).wait()
        base_e = (base_c + b) * CHUNK
        def mul_row(i, b=b, base_e=base_e):
          wb = plsc.load_gather(
              w_v, [jnp.full((L,), 0, jnp.int32) + base_e + i])
          for jj in range(d // L):
            rows_v[b, i, pl.ds(jj * L, L)] = (
                rows_v[b, i, pl.ds(jj * L, L)] * wb)
        plsc.parallel_loop(0, CHUNK, unroll=4)(mul_row)
        for j in range(OPS):
          row = (base_c + b) * OPS + j
          scps.append(
              pltpu.async_copy(rows_v.at[b, pl.ds(j * BURST, BURST)],
                               acc.at[dst_v.at[row]], ssem, add=True))
      for cp in scps:
        cp.wait()

    def outer_body(o, carry):
      do_round(o * nbuf, nbuf)
      return carry

    lax.fori_loop(0, outer, outer_body, 0)
    if tail:
      do_round(outer * nbuf, tail)
    plsc.subcore_barrier()
    pltpu.sync_copy(acc.at[pl.ds(base_n, NODES_PER_TILE)],
                    out_hbm.at[c, pl.ds(base_n, NODES_PER_TILE)])

  return agg


_sc_agg_hidden = _make_sc_aggregate(HIDDEN)
_sc_agg_embed = _make_sc_aggregate(EMBED)


def _mm_body(x_ref, w_ref, o_ref):
  o_ref[...] = jnp.dot(x_ref[...], w_ref[...],
                       preferred_element_type=jnp.float32)


def _fuse_mm_body(p_ref, b_ref, w_ref, o_ref):
  h = jnp.maximum(p_ref[0, :N] + p_ref[1, :N] + b_ref[...], 0.0)
  o_ref[...] = jnp.dot(h, w_ref[...], preferred_element_type=jnp.float32)


def _combine_body(p_ref, b_ref, o_ref):
  o_ref[...] = p_ref[0, :N] + p_ref[1, :N] + b_ref[...]


@jax.jit
def kernel(x, edge_index, edge_weight, W1, b1, W2, b2):
  src3d = edge_index[0].reshape(NC * NS, BURSTS_PER_WORKER, BURST)
  dst3d = edge_index[1].reshape(NC * NS, BURSTS_PER_WORKER, BURST)
  w2d = edge_weight.reshape(NC * NS, EDGES_PER_WORKER)

  h1 = pl.pallas_call(
      _mm_body,
      out_shape=jax.ShapeDtypeStruct((N, HIDDEN), jnp.float32),
  )(x, W1)

  p1 = _sc_agg_hidden(h1, src3d, dst3d, w2d)

  h2 = pl.pallas_call(
      _fuse_mm_body,
      out_shape=jax.ShapeDtypeStruct((N, EMBED), jnp.float32),
  )(p1, b1.reshape(1, HIDDEN), W2)

  p2 = _sc_agg_embed(h2, src3d, dst3d, w2d)

  out = pl.pallas_call(
      _combine_body,
      out_shape=jax.ShapeDtypeStruct((N, EMBED), jnp.float32),
  )(p2, b2.reshape(1, EMBED))
  return out


# speedup vs baseline: 1.0710x; 1.0205x over previous
import functools

import jax
import jax.numpy as jnp
from jax import lax
from jax.experimental import pallas as pl
from jax.experimental.pallas import tpu as pltpu
from jax.experimental.pallas import tpu_sc as plsc

N = 10000
E = 320000
IN_DIM = 128
HIDDEN = 64
EMBED = 32

NC = 2
NS = 16
L = 16

BURST = 40
CHUNK = 200
OPS = CHUNK // BURST
EDGES_PER_WORKER = E // (NC * NS)
BURSTS_PER_WORKER = EDGES_PER_WORKER // BURST
CHUNKS = EDGES_PER_WORKER // CHUNK
NPAD = 10240
NODES_PER_TILE = NPAD // NS


def _make_sc_aggregate(d):
  mesh = plsc.VectorSubcoreMesh(core_axis_name="c", subcore_axis_name="s")
  nbuf = 4 if d > 32 else 8
  outer = CHUNKS // nbuf
  tail = CHUNKS - outer * nbuf

  @functools.partial(
      pl.kernel,
      out_type=jax.ShapeDtypeStruct((NC, NPAD, d), jnp.float32),
      mesh=mesh,
      scratch_types=[
          pltpu.VMEM_SHARED((NPAD, d), jnp.float32),
          pltpu.VMEM((BURSTS_PER_WORKER, BURST), jnp.int32),
          pltpu.VMEM((BURSTS_PER_WORKER, BURST), jnp.int32),
          pltpu.VMEM((EDGES_PER_WORKER,), jnp.float32),
          pltpu.VMEM((nbuf, CHUNK, d), jnp.float32),
          pltpu.SemaphoreType.DMA,
          pltpu.SemaphoreType.DMA,
          pltpu.SemaphoreType.DMA,
      ],
      compiler_params=pltpu.CompilerParams(
          needs_layout_passes=False, use_tc_tiling_on_sc=False),
  )
  def agg(h_hbm, src_hbm, dst_hbm, w_hbm, out_hbm,
          acc, src_v, dst_v, w_v, rows_v, lsem, gsem, ssem):
    c = lax.axis_index("c")
    s = lax.axis_index("s")
    wid = c * NS + s

    lcps = [
        pltpu.async_copy(src_hbm.at[wid], src_v, lsem),
        pltpu.async_copy(dst_hbm.at[wid], dst_v, lsem),
        pltpu.async_copy(w_hbm.at[wid], w_v, lsem),
    ]

    def zero_row(i, carry):
      for j in range(d // L):
        rows_v[0, i, pl.ds(j * L, L)] = jnp.zeros((L,), jnp.float32)
      return carry
    lax.fori_loop(0, CHUNK, zero_row, 0)
    base_n = s * NODES_PER_TILE
    done = 0
    zcps = []
    while done < NODES_PER_TILE:
      step = min(CHUNK, NODES_PER_TILE - done)
      zcps.append(pltpu.async_copy(rows_v.at[0, pl.ds(0, step)],
                                   acc.at[pl.ds(base_n + done, step)], gsem))
      done += step
    for cp in zcps + lcps:
      cp.wait()
    plsc.subcore_barrier()

    last = nbuf - 1

    def gathers_for(base_c, b):
      return [
          pltpu.async_copy(h_hbm.at[src_v.at[(base_c + b) * OPS + j]],
                           rows_v.at[b, pl.ds(j * BURST, BURST)], gsem)
          for j in range(OPS)
      ]

    def process(base_c, b, gcps):
      for cp in gcps:
        cp.wait()
      base_e = (base_c + b) * CHUNK
      def mul_row(i, b=b, base_e=base_e):
        wb = plsc.load_gather(
            w_v, [jnp.full((L,), 0, jnp.int32) + base_e + i])
        for jj in range(d // L):
          rows_v[b, i, pl.ds(jj * L, L)] = (
              rows_v[b, i, pl.ds(jj * L, L)] * wb)
      plsc.parallel_loop(0, CHUNK, unroll=4)(mul_row)
      return [
          pltpu.async_copy(rows_v.at[b, pl.ds(j * BURST, BURST)],
                           acc.at[dst_v.at[(base_c + b) * OPS + j]],
                           ssem, add=True)
          for j in range(OPS)
      ]

    def drain_last_scatter():
      for j in range(OPS):
        pltpu.make_async_copy(rows_v.at[last, pl.ds(j * BURST, BURST)],
                              acc.at[dst_v.at[j]], ssem).wait()

    g0 = [gathers_for(0, b) for b in range(nbuf)]
    s0 = [process(0, b, g0[b]) for b in range(nbuf)]
    for b in range(last):
      for cp in s0[b]:
        cp.wait()

    def outer_body(o, carry):
      base_c = o * nbuf
      gcps = [gathers_for(base_c, b) for b in range(last)]
      drain_last_scatter()
      gcps.append(gathers_for(base_c, last))
      scps = [process(base_c, b, gcps[b]) for b in range(nbuf)]
      for b in range(last):
        for cp in scps[b]:
          cp.wait()
      return carry

    lax.fori_loop(1, outer, outer_body, 0)
    drain_last_scatter()
    if tail:
      gt = [gathers_for(outer * nbuf, b) for b in range(tail)]
      st = [process(outer * nbuf, b, gt[b]) for b in range(tail)]
      for b in range(tail):
        for cp in st[b]:
          cp.wait()
    plsc.subcore_barrier()
    pltpu.sync_copy(acc.at[pl.ds(base_n, NODES_PER_TILE)],
                    out_hbm.at[c, pl.ds(base_n, NODES_PER_TILE)])

  return agg


_sc_agg_hidden = _make_sc_aggregate(HIDDEN)
_sc_agg_embed = _make_sc_aggregate(EMBED)


def _mm_body(x_ref, w_ref, o_ref):
  o_ref[...] = jnp.dot(x_ref[...], w_ref[...],
                       preferred_element_type=jnp.float32)


def _fuse_mm_body(p_ref, b_ref, w_ref, o_ref):
  h = jnp.maximum(p_ref[0, :N] + p_ref[1, :N] + b_ref[...], 0.0)
  o_ref[...] = jnp.dot(h, w_ref[...], preferred_element_type=jnp.float32)


def _combine_body(p_ref, b_ref, o_ref):
  o_ref[...] = p_ref[0, :N] + p_ref[1, :N] + b_ref[...]


@jax.jit
def kernel(x, edge_index, edge_weight, W1, b1, W2, b2):
  src3d = edge_index[0].reshape(NC * NS, BURSTS_PER_WORKER, BURST)
  dst3d = edge_index[1].reshape(NC * NS, BURSTS_PER_WORKER, BURST)
  w2d = edge_weight.reshape(NC * NS, EDGES_PER_WORKER)

  h1 = pl.pallas_call(
      _mm_body,
      out_shape=jax.ShapeDtypeStruct((N, HIDDEN), jnp.float32),
  )(x, W1)

  p1 = _sc_agg_hidden(h1, src3d, dst3d, w2d)

  h2 = pl.pallas_call(
      _fuse_mm_body,
      out_shape=jax.ShapeDtypeStruct((N, EMBED), jnp.float32),
  )(p1, b1.reshape(1, HIDDEN), W2)

  p2 = _sc_agg_embed(h2, src3d, dst3d, w2d)

  out = pl.pallas_call(
      _combine_body,
      out_shape=jax.ShapeDtypeStruct((N, EMBED), jnp.float32),
  )(p2, b2.reshape(1, EMBED))
  return out
